# baseline (speedup 1.0000x reference)
import jax
import jax.numpy as jnp
from jax import lax
from jax.experimental import pallas as pl
from jax.experimental.pallas import tpu as pltpu
from jax.experimental.pallas import tpu_sc as plsc

_BATCH = 16384
_HIST = 20
_DIM = 64
_NC = 2
_NS = 16
_NW = _NC * _NS
_ROWS_W = _BATCH // _NW
_Q = 128
_NQ = _ROWS_W // _Q
_NSTEP = _HIST * _NQ
_HALF = 256


def _gather_body(idx_hbm, table_hbm, out_hbm,
                 idx_vb, off_t, par_t, pstage, tstage, gsem, ssem):
    wid = lax.axis_index("s") * _NC + lax.axis_index("c")
    b0 = wid * _ROWS_W
    lanes = lax.iota(jnp.int32, 16)

    for ch in range(4):
        pltpu.sync_copy(idx_hbm.at[pl.ds(b0 + ch * _Q, _Q)], idx_vb)
        for h in range(_HIST):
            for l in range(8):
                rows = lanes + (16 * l)
                cols = jnp.full((16,), h, jnp.int32)
                v = plsc.load_gather(idx_vb, [rows, cols])
                off_t[h, pl.ds(ch * _Q + 16 * l, 16)] = v >> 1
                par_t[h, pl.ds(ch * _Q + 16 * l, 16)] = (v & 1) * _DIM

    def fire_gather(t):
        h = t >> 2
        q = lax.rem(t, 4)
        rung = lax.rem(t, 3)
        pltpu.async_copy(
            table_hbm.at[off_t.at[h, pl.ds(q * _Q, _Q)]],
            pstage.at[rung], gsem.at[rung])

    def wait_gather(t):
        h = t >> 2
        q = lax.rem(t, 4)
        rung = lax.rem(t, 3)
        pltpu.make_async_copy(
            table_hbm.at[off_t.at[h, pl.ds(q * _Q, _Q)]],
            pstage.at[rung], gsem.at[rung]).wait()

    def write_half(h, hf, buf):
        pltpu.async_copy(
            tstage.at[buf],
            out_hbm.at[h].at[:, pl.ds(b0 + hf * _HALF, _HALF)],
            ssem.at[buf])

    def wait_write(h, hf, buf):
        pltpu.make_async_copy(
            tstage.at[buf],
            out_hbm.at[h].at[:, pl.ds(b0 + hf * _HALF, _HALF)],
            ssem.at[buf]).wait()

    fire_gather(0)
    fire_gather(1)

    def step(t, carry):
        h = t >> 2
        q = lax.rem(t, 4)
        rung = lax.rem(t, 3)
        buf = lax.rem(t >> 1, 2)
        wait_gather(t)

        @pl.when(t + 2 < _NSTEP)
        def _():
            fire_gather(t + 2)

        @pl.when((lax.rem(t, 2) == 0) & (t >= 4))
        def _():
            wait_write(h - 1, q >> 1, buf)

        colb = lax.rem(q, 2) * _Q
        for g in range(8):
            b_rows = lanes + 16 * g
            par_vec = par_t[h, pl.ds(q * _Q + 16 * g, 16)]

            def col(c, c2):
                v = plsc.load_gather(pstage.at[rung], [b_rows, par_vec + c])
                tstage[buf, c, pl.ds(colb + 16 * g, 16)] = v
                return c2

            lax.fori_loop(0, _DIM, col, 0)

        @pl.when(lax.rem(t, 2) == 1)
        def _():
            write_half(h, q >> 1, buf)

        return carry

    lax.fori_loop(0, _NSTEP, step, 0)

    wait_write(_HIST - 1, 0, 0)
    wait_write(_HIST - 1, 1, 1)


@jax.jit
def kernel(indices, table):
    idx = indices.astype(jnp.int32)
    t2 = table.reshape(table.shape[0] // 2, 2 * table.shape[1])
    mesh = plsc.VectorSubcoreMesh(core_axis_name="c", subcore_axis_name="s")
    out = pl.kernel(
        _gather_body,
        out_type=jax.ShapeDtypeStruct((_HIST, _DIM, _BATCH), jnp.float32),
        mesh=mesh,
        scratch_types=[
            pltpu.VMEM((_Q, _HIST), jnp.int32),
            pltpu.VMEM((_HIST, _ROWS_W), jnp.int32),
            pltpu.VMEM((_HIST, _ROWS_W), jnp.int32),
            pltpu.VMEM((3, _Q, 2 * _DIM), jnp.float32),
            pltpu.VMEM((2, _DIM, _HALF), jnp.float32),
            pltpu.SemaphoreType.DMA((3,)),
            pltpu.SemaphoreType.DMA((2,)),
        ],
        compiler_params=pltpu.CompilerParams(
            use_tc_tiling_on_sc=True, needs_layout_passes=False),
    )(idx, t2)
    return out.transpose(2, 0, 1)
